# baseline (speedup 1.0000x reference)
import jax
import jax.numpy as jnp
from jax import lax
from jax.experimental import pallas as pl
from jax.experimental.pallas import tpu as pltpu
from jax.experimental.pallas import tpu_sc as plsc

N_DATA = 1000000
BATCH = 16384
NC = 2
NS = 16
L = 16
NW = NC * NS
B_PER_W = BATCH // NW
GROUPS = B_PER_W // L
CHUNK = 128
CHUNKS = B_PER_W // CHUNK


def _sc_body(wflat, rflat, idx_hbm, out_hbm, idx_v, sidx_v, wsoa, rsoa,
             oaos, sem_w, sem_r):
    wid = lax.axis_index("s") * NC + lax.axis_index("c")
    row0 = wid * B_PER_W

    pltpu.sync_copy(idx_hbm.at[pl.ds(wid * CHUNKS, CHUNKS)], idx_v)

    for j in range(CHUNKS):
        def scale(g, carry):
            v = idx_v[j, pl.ds(g * L, L)]
            for c in range(3):
                sidx_v[12 * j + c, pl.ds(g * L, L)] = v + (c * N_DATA)
            for c in range(9):
                sidx_v[12 * j + 3 + c, pl.ds(g * L, L)] = v + (c * N_DATA)
            return carry
        lax.fori_loop(0, CHUNK // L, scale, 0)

    cps = []
    for j in range(CHUNKS):
        for c in range(3):
            cps.append(pltpu.async_copy(
                wflat.at[sidx_v.at[12 * j + c]],
                wsoa.at[pl.ds(c * B_PER_W + j * CHUNK, CHUNK)], sem_w))
        for c in range(9):
            cps.append(pltpu.async_copy(
                rflat.at[sidx_v.at[12 * j + 3 + c]],
                rsoa.at[pl.ds(c * B_PER_W + j * CHUNK, CHUNK)], sem_r))
    for cp in cps:
        cp.wait()

    iota = lax.iota(jnp.int32, L)
    cols9 = [jnp.full((L,), c, jnp.int32) for c in range(9)]

    def group(g, carry):
        s = g * L
        rows = s + iota

        wx = wsoa[pl.ds(s, L)]
        wy = wsoa[pl.ds(B_PER_W + s, L)]
        wz = wsoa[pl.ds(2 * B_PER_W + s, L)]
        r = [rsoa[pl.ds(c * B_PER_W + s, L)] for c in range(9)]

        xx = wx * wx
        yy = wy * wy
        zz = wz * wz
        u = xx + yy + zz

        a = 1.0 + u * (-1.0 / 6.0 + u * (1.0 / 120.0 + u * (-1.0 / 5040.0)))
        b = 0.5 + u * (-1.0 / 24.0 + u * (1.0 / 720.0 + u * (-1.0 / 40320.0)))

        bxy = b * (wx * wy)
        bxz = b * (wx * wz)
        byz = b * (wy * wz)
        ax = a * wx
        ay = a * wy
        az = a * wz

        d00 = 1.0 - b * (yy + zz)
        d01 = bxy - az
        d02 = bxz + ay
        d10 = bxy + az
        d11 = 1.0 - b * (xx + zz)
        d12 = byz - ax
        d20 = bxz - ay
        d21 = byz + ax
        d22 = 1.0 - b * (xx + yy)
        d = (d00, d01, d02, d10, d11, d12, d20, d21, d22)

        for i in range(3):
            for jj in range(3):
                o = (d[3 * i] * r[jj] + d[3 * i + 1] * r[3 + jj]
                     + d[3 * i + 2] * r[6 + jj])
                plsc.store_scatter(oaos, [rows, cols9[3 * i + jj]], o)
        return carry

    lax.fori_loop(0, GROUPS, group, 0)

    pltpu.sync_copy(oaos, out_hbm.at[pl.ds(row0, B_PER_W)])


@jax.jit
def _run(wflat, rflat, idx2d):
    kern = pl.kernel(
        _sc_body,
        out_type=jax.ShapeDtypeStruct((BATCH, 9), jnp.float32),
        mesh=plsc.VectorSubcoreMesh(
            core_axis_name="c", subcore_axis_name="s",
            num_cores=NC, num_subcores=NS),
        scratch_types=[
            pltpu.VMEM((CHUNKS, CHUNK), jnp.int32),
            pltpu.VMEM((12 * CHUNKS, CHUNK), jnp.int32),
            pltpu.VMEM((3 * B_PER_W,), jnp.float32),
            pltpu.VMEM((9 * B_PER_W,), jnp.float32),
            pltpu.VMEM((B_PER_W, 9), jnp.float32),
            pltpu.SemaphoreType.DMA,
            pltpu.SemaphoreType.DMA,
        ],
        compiler_params=pltpu.CompilerParams(
            needs_layout_passes=False, use_tc_tiling_on_sc=False),
    )
    return kern(wflat, rflat, idx2d)


def kernel(perturbations_w, rotations, idx):
    wflat = jnp.stack(
        [perturbations_w[:, c] for c in range(3)]).reshape(3 * N_DATA)
    rflat = jnp.stack(
        [rotations[:, i, j] for i in range(3) for j in range(3)]
    ).reshape(9 * N_DATA)
    idx2d = idx.astype(jnp.int32).reshape(BATCH // CHUNK, CHUNK)
    out = _run(wflat, rflat, idx2d)
    return out.reshape(BATCH, 3, 3)


# speedup vs baseline: 1.3122x; 1.3122x over previous
import jax
import jax.numpy as jnp
from jax import lax
from jax.experimental import pallas as pl
from jax.experimental.pallas import tpu as pltpu
from jax.experimental.pallas import tpu_sc as plsc

N_DATA = 1000000
BATCH = 16384
NC = 2
NS = 16
L = 16
NW = NC * NS
B_PER_W = BATCH // NW
GROUPS = B_PER_W // L
CHUNK = 128
CHUNKS = B_PER_W // CHUNK


def _sc_body(wflat, rflat, idx_hbm, out_hbm, idx_v, sidx_v, wsoa, rsoa,
             oaos, sem_w, sem_r):
    wid = lax.axis_index("s") * NC + lax.axis_index("c")
    row0 = wid * B_PER_W

    pltpu.sync_copy(idx_hbm.at[pl.ds(wid * CHUNKS, CHUNKS)], idx_v)

    for j in range(CHUNKS):
        def scale(g, carry):
            v = idx_v[j, pl.ds(g * L, L)]
            for c in range(3):
                sidx_v[12 * j + c, pl.ds(g * L, L)] = v + (c * N_DATA)
            for c in range(9):
                sidx_v[12 * j + 3 + c, pl.ds(g * L, L)] = v + (c * N_DATA)
            return carry
        lax.fori_loop(0, CHUNK // L, scale, 0)

    cps = []
    for j in range(CHUNKS):
        for c in range(3):
            cps.append(pltpu.async_copy(
                wflat.at[sidx_v.at[12 * j + c]],
                wsoa.at[pl.ds(c * B_PER_W + j * CHUNK, CHUNK)], sem_w))
        for c in range(9):
            cps.append(pltpu.async_copy(
                rflat.at[sidx_v.at[12 * j + 3 + c]],
                rsoa.at[pl.ds(c * B_PER_W + j * CHUNK, CHUNK)], sem_r))
    for cp in cps:
        cp.wait()

    iota = lax.iota(jnp.int32, L)
    cols9 = [jnp.full((L,), c, jnp.int32) for c in range(9)]

    def group(g, carry):
        s = g * L
        rows = s + iota

        wx = wsoa[pl.ds(s, L)]
        wy = wsoa[pl.ds(B_PER_W + s, L)]
        wz = wsoa[pl.ds(2 * B_PER_W + s, L)]
        r = [rsoa[pl.ds(c * B_PER_W + s, L)] for c in range(9)]

        xx = wx * wx
        yy = wy * wy
        zz = wz * wz
        u = xx + yy + zz

        a = 1.0 + u * (-1.0 / 6.0 + u * (1.0 / 120.0 + u * (-1.0 / 5040.0)))
        b = 0.5 + u * (-1.0 / 24.0 + u * (1.0 / 720.0 + u * (-1.0 / 40320.0)))

        bxy = b * (wx * wy)
        bxz = b * (wx * wz)
        byz = b * (wy * wz)
        ax = a * wx
        ay = a * wy
        az = a * wz

        d00 = 1.0 - b * (yy + zz)
        d01 = bxy - az
        d02 = bxz + ay
        d10 = bxy + az
        d11 = 1.0 - b * (xx + zz)
        d12 = byz - ax
        d20 = bxz - ay
        d21 = byz + ax
        d22 = 1.0 - b * (xx + yy)
        d = (d00, d01, d02, d10, d11, d12, d20, d21, d22)

        for i in range(3):
            for jj in range(3):
                o = (d[3 * i] * r[jj] + d[3 * i + 1] * r[3 + jj]
                     + d[3 * i + 2] * r[6 + jj])
                plsc.store_scatter(oaos, [rows, cols9[3 * i + jj]], o)
        return carry

    lax.fori_loop(0, GROUPS, group, 0)

    pltpu.sync_copy(oaos, out_hbm.at[pl.ds(row0, B_PER_W)])


@jax.jit
def _run(wflat, rflat, idx2d):
    kern = pl.kernel(
        _sc_body,
        out_type=jax.ShapeDtypeStruct((BATCH, 9), jnp.float32),
        mesh=plsc.VectorSubcoreMesh(
            core_axis_name="c", subcore_axis_name="s",
            num_cores=NC, num_subcores=NS),
        scratch_types=[
            pltpu.VMEM((CHUNKS, CHUNK), jnp.int32),
            pltpu.VMEM((12 * CHUNKS, CHUNK), jnp.int32),
            pltpu.VMEM((3 * B_PER_W,), jnp.float32),
            pltpu.VMEM((9 * B_PER_W,), jnp.float32),
            pltpu.VMEM((B_PER_W, 9), jnp.float32),
            pltpu.SemaphoreType.DMA,
            pltpu.SemaphoreType.DMA,
        ],
        compiler_params=pltpu.CompilerParams(
            needs_layout_passes=False, use_tc_tiling_on_sc=False),
    )
    return kern(wflat, rflat, idx2d)


def kernel(perturbations_w, rotations, idx):
    wflat = perturbations_w.T.reshape(3 * N_DATA)
    rflat = jnp.transpose(rotations, (1, 2, 0)).reshape(9 * N_DATA)
    idx2d = idx.astype(jnp.int32).reshape(BATCH // CHUNK, CHUNK)
    out = _run(wflat, rflat, idx2d)
    return out.reshape(BATCH, 3, 3)


# speedup vs baseline: 4.4543x; 3.3944x over previous
import jax
import jax.numpy as jnp
from jax import lax
from jax.experimental import pallas as pl
from jax.experimental.pallas import tpu as pltpu
from jax.experimental.pallas import tpu_sc as plsc

N_DATA = 1000000
BATCH = 16384
NC = 2
NS = 16
L = 16
NW = NC * NS
B_PER_W = BATCH // NW
GROUPS = B_PER_W // L
CHUNK = 128
CHUNKS = B_PER_W // CHUNK

_MESH = plsc.VectorSubcoreMesh(
    core_axis_name="c", subcore_axis_name="s",
    num_cores=NC, num_subcores=NS)
_PARAMS = pltpu.CompilerParams(
    needs_layout_passes=False, use_tc_tiling_on_sc=False)


def _delta_body(wt0, wt1, wt2, idx_hbm, delta_hbm, idx_v, wsoa, daos, sem_w):
    wtabs = (wt0, wt1, wt2)
    wid = lax.axis_index("s") * NC + lax.axis_index("c")
    row0 = wid * B_PER_W

    pltpu.sync_copy(idx_hbm.at[pl.ds(wid * CHUNKS, CHUNKS)], idx_v)

    cps = []
    for j in range(CHUNKS):
        ids = idx_v.at[j]
        for c in range(3):
            cps.append(pltpu.async_copy(
                wtabs[c].at[ids],
                wsoa.at[pl.ds(c * B_PER_W + j * CHUNK, CHUNK)], sem_w))
    for cp in cps:
        cp.wait()

    iota = lax.iota(jnp.int32, L)
    cols9 = [jnp.full((L,), c, jnp.int32) for c in range(9)]

    def group(g, carry):
        s = g * L
        rows = s + iota

        wx = wsoa[pl.ds(s, L)]
        wy = wsoa[pl.ds(B_PER_W + s, L)]
        wz = wsoa[pl.ds(2 * B_PER_W + s, L)]

        xx = wx * wx
        yy = wy * wy
        zz = wz * wz
        u = xx + yy + zz

        a = 1.0 + u * (-1.0 / 6.0 + u * (1.0 / 120.0 + u * (-1.0 / 5040.0)))
        b = 0.5 + u * (-1.0 / 24.0 + u * (1.0 / 720.0 + u * (-1.0 / 40320.0)))

        bxy = b * (wx * wy)
        bxz = b * (wx * wz)
        byz = b * (wy * wz)
        ax = a * wx
        ay = a * wy
        az = a * wz

        d = (1.0 - b * (yy + zz), bxy - az, bxz + ay,
             bxy + az, 1.0 - b * (xx + zz), byz - ax,
             bxz - ay, byz + ax, 1.0 - b * (xx + yy))
        for c in range(9):
            plsc.store_scatter(daos, [rows, cols9[c]], d[c])
        return carry

    lax.fori_loop(0, GROUPS, group, 0)

    pltpu.sync_copy(daos, delta_hbm.at[pl.ds(row0, B_PER_W)])


def _matmul_body(*refs):
    (r0, r1, r2, r3, r4, r5, r6, r7, r8, idx_hbm, delta_hbm,
     out_hbm, idx_v, rsoa, dv, oaos, sem_r, sem_d) = refs
    rtabs = (r0, r1, r2, r3, r4, r5, r6, r7, r8)

    wid = lax.axis_index("s") * NC + lax.axis_index("c")
    row0 = wid * B_PER_W

    pltpu.sync_copy(idx_hbm.at[pl.ds(wid * CHUNKS, CHUNKS)], idx_v)

    cps = [pltpu.async_copy(delta_hbm.at[pl.ds(row0, B_PER_W)], dv, sem_d)]
    for j in range(CHUNKS):
        ids = idx_v.at[j]
        for c in range(9):
            cps.append(pltpu.async_copy(
                rtabs[c].at[ids],
                rsoa.at[pl.ds(c * B_PER_W + j * CHUNK, CHUNK)], sem_r))
    for cp in cps:
        cp.wait()

    iota = lax.iota(jnp.int32, L)
    cols9 = [jnp.full((L,), c, jnp.int32) for c in range(9)]

    def group(g, carry):
        s = g * L
        rows = s + iota

        d = [plsc.load_gather(dv, [rows, cols9[c]]) for c in range(9)]
        r = [rsoa[pl.ds(c * B_PER_W + s, L)] for c in range(9)]

        for i in range(3):
            for jj in range(3):
                o = (d[3 * i] * r[jj] + d[3 * i + 1] * r[3 + jj]
                     + d[3 * i + 2] * r[6 + jj])
                plsc.store_scatter(oaos, [rows, cols9[3 * i + jj]], o)
        return carry

    lax.fori_loop(0, GROUPS, group, 0)

    pltpu.sync_copy(oaos, out_hbm.at[pl.ds(row0, B_PER_W)])


@jax.jit
def _run(wcols, rcols, idx2d):
    k1 = pl.kernel(
        _delta_body,
        out_type=jax.ShapeDtypeStruct((BATCH, 9), jnp.float32),
        mesh=_MESH,
        scratch_types=[
            pltpu.VMEM((CHUNKS, CHUNK), jnp.int32),
            pltpu.VMEM((3 * B_PER_W,), jnp.float32),
            pltpu.VMEM((B_PER_W, 9), jnp.float32),
            pltpu.SemaphoreType.DMA,
        ],
        compiler_params=_PARAMS,
    )
    delta = k1(*wcols, idx2d)

    k2 = pl.kernel(
        _matmul_body,
        out_type=jax.ShapeDtypeStruct((BATCH, 9), jnp.float32),
        mesh=_MESH,
        scratch_types=[
            pltpu.VMEM((CHUNKS, CHUNK), jnp.int32),
            pltpu.VMEM((9 * B_PER_W,), jnp.float32),
            pltpu.VMEM((B_PER_W, 9), jnp.float32),
            pltpu.VMEM((B_PER_W, 9), jnp.float32),
            pltpu.SemaphoreType.DMA,
            pltpu.SemaphoreType.DMA,
        ],
        compiler_params=_PARAMS,
    )
    return k2(*rcols, idx2d, delta)


def kernel(perturbations_w, rotations, idx):
    wcols = [perturbations_w[:, c] for c in range(3)]
    rcols = [rotations[:, i, j] for i in range(3) for j in range(3)]
    idx2d = idx.astype(jnp.int32).reshape(BATCH // CHUNK, CHUNK)
    out = _run(wcols, rcols, idx2d)
    return out.reshape(BATCH, 3, 3)
